# baseline (speedup 1.0000x reference)
import functools

import jax
import jax.numpy as jnp
from jax.experimental import pallas as pl
from jax.experimental.pallas import tpu as pltpu

BASE_SPARSITY = 0.1


def _router_block(x_ref, wc_ref, bias_ref, pat_ref, out_ref, misc_ref, *, k, p):
    xb = x_ref[...]
    logits_all = jnp.dot(xb, wc_ref[...], preferred_element_type=jnp.float32)
    logits_all = logits_all + bias_ref[...]
    T = xb.shape[0]
    lane = jax.lax.broadcasted_iota(jnp.int32, logits_all.shape, 1)
    neg_inf = jnp.float32(-jnp.inf)
    sel_logits = jnp.where(lane < p, logits_all, neg_inf)
    m = jnp.max(sel_logits, axis=1, keepdims=True)
    e = jnp.where(lane < p, jnp.exp(sel_logits - m), 0.0)
    w = e / jnp.sum(e, axis=1, keepdims=True)
    intensity = jax.nn.sigmoid(logits_all[:, p:p + 1])

    g = jnp.dot(w, pat_ref[...], preferred_element_type=jnp.float32)
    g = g * intensity
    bits = jax.lax.bitcast_convert_type(g, jnp.int32) & jnp.int32(0x7FFFFFFF)

    def body(_, carry):
        lo, hi = carry
        mid = lo + jax.lax.shift_right_logical(hi - lo, 1)
        cnt = jnp.count_nonzero(bits >= mid, axis=1, keepdims=True)
        ge = cnt >= k
        return jnp.where(ge, mid, lo), jnp.where(ge, hi, mid)

    lo0 = jnp.zeros((T, 1), jnp.int32)
    hi0 = jnp.full((T, 1), jnp.int32(0x7F800001))
    lo, _ = jax.lax.fori_loop(0, 31, body, (lo0, hi0), unroll=8)

    out_ref[...] = jnp.where(bits >= lo, g, 0.0)

    ent = -jnp.sum(w * jnp.log(w + 1e-08), axis=1, keepdims=True)
    misc_ref[...] = jnp.concatenate(
        [w[:, :p], ent, intensity, jnp.zeros((T, 6), jnp.float32)], axis=1)


@functools.partial(jax.jit, static_argnames=())
def kernel(x, flow_patterns, sel_w, sel_b, int_w, int_b):
    B, S, D_IN = x.shape
    P, O, I = flow_patterns.shape
    N = B * S
    OI = O * I
    base_k = max(1, int(OI * BASE_SPARSITY))
    k = min(base_k, OI // 2)

    x2 = x.reshape(N, D_IN)
    wc = jnp.zeros((D_IN, 128), jnp.float32)
    wc = wc.at[:, :P].set(sel_w.T).at[:, P:P + 1].set(int_w.T)
    bias = jnp.zeros((1, 128), jnp.float32)
    bias = bias.at[0, :P].set(sel_b).at[0, P].set(int_b[0])
    pat = jnp.zeros((128, OI), jnp.float32)
    pat = pat.at[:P, :].set(flow_patterns.reshape(P, OI))

    T = 256
    G = N // T
    out, misc = pl.pallas_call(
        functools.partial(_router_block, k=k, p=P),
        grid=(G,),
        in_specs=[
            pl.BlockSpec((T, D_IN), lambda i: (i, 0)),
            pl.BlockSpec((D_IN, 128), lambda i: (0, 0)),
            pl.BlockSpec((1, 128), lambda i: (0, 0)),
            pl.BlockSpec((128, OI), lambda i: (0, 0)),
        ],
        out_specs=[
            pl.BlockSpec((T, OI), lambda i: (i, 0)),
            pl.BlockSpec((T, 16), lambda i: (i, 0)),
        ],
        out_shape=[
            jax.ShapeDtypeStruct((N, OI), jnp.float32),
            jax.ShapeDtypeStruct((N, 16), jnp.float32),
        ],
        compiler_params=pltpu.CompilerParams(
            dimension_semantics=("parallel",)),
    )(x2, wc, bias, pat)

    pattern_entropy = jnp.mean(misc[:, P])
    flow_intensity_mean = jnp.mean(misc[:, P + 1])
    pattern_diversity = jnp.std(jnp.mean(misc[:, :P], axis=0), ddof=1)
    return (out.reshape(B, S, O, I), pattern_entropy, flow_intensity_mean,
            pattern_diversity)


# speedup vs baseline: 1.0580x; 1.0580x over previous
import functools

import jax
import jax.numpy as jnp
from jax.experimental import pallas as pl
from jax.experimental.pallas import tpu as pltpu

BASE_SPARSITY = 0.1


def _router_block(x_ref, wc_ref, bias_ref, pat_ref, out_ref, misc_ref, *, k, p):
    xb = x_ref[...]
    logits_all = jnp.dot(xb, wc_ref[...], preferred_element_type=jnp.float32)
    logits_all = logits_all + bias_ref[...]
    T = xb.shape[0]
    lane = jax.lax.broadcasted_iota(jnp.int32, logits_all.shape, 1)
    neg_inf = jnp.float32(-jnp.inf)
    sel_logits = jnp.where(lane < p, logits_all, neg_inf)
    m = jnp.max(sel_logits, axis=1, keepdims=True)
    e = jnp.where(lane < p, jnp.exp(sel_logits - m), 0.0)
    w = e / jnp.sum(e, axis=1, keepdims=True)
    intensity = jax.nn.sigmoid(logits_all[:, p:p + 1])

    g = jnp.dot(w, pat_ref[...], preferred_element_type=jnp.float32)
    g = g * intensity
    bits = jax.lax.bitcast_convert_type(g, jnp.int32) & jnp.int32(0x7FFFFFFF)

    def body(_, carry):
        lo, hi = carry
        mid = lo + jax.lax.shift_right_logical(hi - lo, 1)
        cnt = jnp.count_nonzero(bits >= mid, axis=1, keepdims=True)
        ge = cnt >= k
        return jnp.where(ge, mid, lo), jnp.where(ge, hi, mid)

    lo0 = jnp.zeros((T, 1), jnp.int32)
    hi0 = jnp.full((T, 1), jnp.int32(0x7F800001))
    lo, _ = jax.lax.fori_loop(0, 31, body, (lo0, hi0), unroll=8)

    out_ref[...] = jnp.where(bits >= lo, g, 0.0)

    ent = -jnp.sum(w * jnp.log(w + 1e-08), axis=1, keepdims=True)
    misc_ref[...] = jnp.concatenate(
        [w[:, :p], ent, intensity, jnp.zeros((T, 6), jnp.float32)], axis=1)


@functools.partial(jax.jit, static_argnames=())
def kernel(x, flow_patterns, sel_w, sel_b, int_w, int_b):
    B, S, D_IN = x.shape
    P, O, I = flow_patterns.shape
    N = B * S
    OI = O * I
    base_k = max(1, int(OI * BASE_SPARSITY))
    k = min(base_k, OI // 2)

    x2 = x.reshape(N, D_IN)
    wc = jnp.zeros((D_IN, 128), jnp.float32)
    wc = wc.at[:, :P].set(sel_w.T).at[:, P:P + 1].set(int_w.T)
    bias = jnp.zeros((1, 128), jnp.float32)
    bias = bias.at[0, :P].set(sel_b).at[0, P].set(int_b[0])
    pat = jnp.zeros((128, OI), jnp.float32)
    pat = pat.at[:P, :].set(flow_patterns.reshape(P, OI))

    T = 512
    G = N // T
    out, misc = pl.pallas_call(
        functools.partial(_router_block, k=k, p=P),
        grid=(G,),
        in_specs=[
            pl.BlockSpec((T, D_IN), lambda i: (i, 0)),
            pl.BlockSpec((D_IN, 128), lambda i: (0, 0)),
            pl.BlockSpec((1, 128), lambda i: (0, 0)),
            pl.BlockSpec((128, OI), lambda i: (0, 0)),
        ],
        out_specs=[
            pl.BlockSpec((T, OI), lambda i: (i, 0)),
            pl.BlockSpec((T, 16), lambda i: (i, 0)),
        ],
        out_shape=[
            jax.ShapeDtypeStruct((N, OI), jnp.float32),
            jax.ShapeDtypeStruct((N, 16), jnp.float32),
        ],
        compiler_params=pltpu.CompilerParams(
            dimension_semantics=("parallel",)),
    )(x2, wc, bias, pat)

    pattern_entropy = jnp.mean(misc[:, P])
    flow_intensity_mean = jnp.mean(misc[:, P + 1])
    pattern_diversity = jnp.std(jnp.mean(misc[:, :P], axis=0), ddof=1)
    return (out.reshape(B, S, O, I), pattern_entropy, flow_intensity_mean,
            pattern_diversity)
